# baseline (speedup 1.0000x reference)
import functools

import jax
import jax.numpy as jnp
from jax import lax
from jax.experimental import pallas as pl
from jax.experimental.pallas import tpu as pltpu
from jax.experimental.pallas import tpu_sc as plsc


def _sc_gather_body(per_worker, table_hbm, idx_hbm, out_hbm,
                    idx_v, rows_v, sem):
    info = plsc.get_sparse_core_info()
    nc = info.num_cores
    wid = lax.axis_index("s") * nc + lax.axis_index("c")
    base = wid * per_worker
    seq = idx_hbm.shape[1]
    row = base // seq
    col = base % seq
    pltpu.sync_copy(idx_hbm.at[row, pl.ds(col, per_worker)], idx_v)
    pltpu.async_copy(table_hbm.at[idx_v], rows_v, sem).wait()
    pltpu.sync_copy(rows_v, out_hbm.at[pl.ds(base, per_worker)])


def _sc_gather(table, idx):
    n_tokens = idx.shape[0] * idx.shape[1]
    emb = table.shape[1]
    info = plsc.get_sparse_core_info()
    n_workers = info.num_cores * info.num_subcores
    per_worker = n_tokens // n_workers
    assert n_tokens % (8 * n_workers) == 0 and idx.shape[1] % per_worker == 0
    mesh = plsc.VectorSubcoreMesh(core_axis_name="c", subcore_axis_name="s")
    body = functools.partial(_sc_gather_body, per_worker)
    return pl.kernel(
        body,
        out_type=jax.ShapeDtypeStruct((n_tokens, emb), jnp.float32),
        mesh=mesh,
        scratch_types=[
            pltpu.VMEM((per_worker,), jnp.int32),
            pltpu.VMEM((per_worker, emb), jnp.float32),
            pltpu.SemaphoreType.DMA,
        ],
    )(table, idx)


def _proj_body(t_ref, pos_ref, w_ref, out_ref, h_ref):
    j = pl.program_id(0)
    s = pl.program_id(1)

    @pl.when((j == 0) & (s == 0))
    def _():
        reps = t_ref.shape[0] // pos_ref.shape[0]
        p = jnp.concatenate([pos_ref[...]] * reps, axis=0)
        h_ref[...] = (t_ref[...] + p).astype(jnp.bfloat16)

    seq = pos_ref.shape[0]
    sub = out_ref.shape[2]
    w = w_ref[...].astype(jnp.bfloat16)
    for b in range(t_ref.shape[0] // seq):
        out_ref[:, b, :] = lax.dot_general(
            w, h_ref[pl.ds(b * seq + s * sub, sub), :],
            dimension_numbers=(((1,), (1,)), ((), ())),
            preferred_element_type=jnp.float32,
            precision=lax.Precision.DEFAULT,
        )


def _projection(t, pos, w_out, block_n, seq_split):
    m, emb = t.shape
    seq = pos.shape[0]
    batch = m // seq
    vocab = w_out.shape[0]
    sub = seq // seq_split
    grid = (pl.cdiv(vocab, block_n), seq_split)
    return pl.pallas_call(
        _proj_body,
        grid=grid,
        in_specs=[
            pl.BlockSpec((m, emb), lambda j, s: (0, 0)),
            pl.BlockSpec(pos.shape, lambda j, s: (0, 0)),
            pl.BlockSpec((block_n, emb), lambda j, s: (j, 0)),
        ],
        out_specs=pl.BlockSpec((block_n, batch, sub), lambda j, s: (j, 0, s)),
        out_shape=jax.ShapeDtypeStruct((vocab, batch, seq), jnp.float32),
        scratch_shapes=[pltpu.VMEM((m, emb), jnp.bfloat16)],
    )(t, pos, w_out)


def kernel(x, tok_emb, pos_emb, W_out):
    t = _sc_gather(tok_emb, x.astype(jnp.int32))
    logits_t = _projection(t, pos_emb, W_out, block_n=1536, seq_split=4)
    return jnp.transpose(logits_t, (1, 2, 0))


# speedup vs baseline: 1.2270x; 1.2270x over previous
import functools

import jax
import jax.numpy as jnp
from jax import lax
from jax.experimental import pallas as pl
from jax.experimental.pallas import tpu as pltpu
from jax.experimental.pallas import tpu_sc as plsc


def _sc_gather_body(per_worker, table_hbm, idx_hbm, out_hbm,
                    idx_v, rows_v, sem):
    info = plsc.get_sparse_core_info()
    nc = info.num_cores
    wid = lax.axis_index("s") * nc + lax.axis_index("c")
    base = wid * per_worker
    seq = idx_hbm.shape[1]
    row = base // seq
    col = base % seq
    pltpu.sync_copy(idx_hbm.at[row, pl.ds(col, per_worker)], idx_v)
    pltpu.async_copy(table_hbm.at[idx_v], rows_v, sem).wait()
    pltpu.sync_copy(rows_v, out_hbm.at[pl.ds(base, per_worker)])


def _sc_gather(table, idx):
    n_tokens = idx.shape[0] * idx.shape[1]
    emb = table.shape[1]
    info = plsc.get_sparse_core_info()
    n_workers = info.num_cores * info.num_subcores
    per_worker = n_tokens // n_workers
    assert n_tokens % (8 * n_workers) == 0 and idx.shape[1] % per_worker == 0
    mesh = plsc.VectorSubcoreMesh(core_axis_name="c", subcore_axis_name="s")
    body = functools.partial(_sc_gather_body, per_worker)
    return pl.kernel(
        body,
        out_type=jax.ShapeDtypeStruct((n_tokens, emb), jnp.float32),
        mesh=mesh,
        scratch_types=[
            pltpu.VMEM((per_worker,), jnp.int32),
            pltpu.VMEM((per_worker, emb), jnp.float32),
            pltpu.SemaphoreType.DMA,
        ],
    )(table, idx)


def _proj_body(t_ref, pos_ref, w_ref, out_ref, h_ref):
    j = pl.program_id(0)
    s = pl.program_id(1)

    @pl.when((j == 0) & (s == 0))
    def _():
        reps = t_ref.shape[0] // pos_ref.shape[0]
        p = jnp.concatenate([pos_ref[...]] * reps, axis=0)
        h_ref[...] = (t_ref[...] + p).astype(jnp.bfloat16)

    seq = pos_ref.shape[0]
    sub = out_ref.shape[2]
    w = w_ref[...].astype(jnp.bfloat16)
    for b in range(t_ref.shape[0] // seq):
        out_ref[:, b, :] = lax.dot_general(
            w, h_ref[pl.ds(b * seq + s * sub, sub), :],
            dimension_numbers=(((1,), (1,)), ((), ())),
            preferred_element_type=jnp.float32,
            precision=lax.Precision.DEFAULT,
        )


def _projection(t, pos, w_out, block_n, seq_split):
    m, emb = t.shape
    seq = pos.shape[0]
    batch = m // seq
    vocab = w_out.shape[0]
    sub = seq // seq_split
    grid = (pl.cdiv(vocab, block_n), seq_split)
    return pl.pallas_call(
        _proj_body,
        grid=grid,
        in_specs=[
            pl.BlockSpec((m, emb), lambda j, s: (0, 0)),
            pl.BlockSpec(pos.shape, lambda j, s: (0, 0)),
            pl.BlockSpec((block_n, emb), lambda j, s: (j, 0)),
        ],
        out_specs=pl.BlockSpec((block_n, batch, sub), lambda j, s: (j, 0, s)),
        out_shape=jax.ShapeDtypeStruct((vocab, batch, seq), jnp.float32),
        scratch_shapes=[pltpu.VMEM((m, emb), jnp.bfloat16)],
    )(t, pos, w_out)


def kernel(x, tok_emb, pos_emb, W_out):
    t = _sc_gather(tok_emb, x.astype(jnp.int32))
    logits_t = _projection(t, pos_emb, W_out, block_n=1536, seq_split=2)
    return jnp.transpose(logits_t, (1, 2, 0))
